# baseline (speedup 1.0000x reference)
import functools
import math

import jax
import jax.numpy as jnp
from jax.experimental import pallas as pl

_CT1 = 0.02
_CT2 = 0.02
_H = 1.02
_NEG_INF = float("-inf")


def _phase_a_body(aw_ref, as_ref, nb_ref, m_ref, lsew_ref, lses_ref,
                  lqbn_ref, tgt_ref, lmax_ref, *, k_steps, c, log_ct1):
    aw = aw_ref[...]
    awmax = jnp.max(aw, axis=1, keepdims=True)
    ew = jnp.exp(aw - awmax)
    sw = jnp.sum(ew, axis=1, keepdims=True)
    weak = ew / sw
    lsew = awmax + jnp.log(sw)
    m = awmax - lsew

    asb = as_ref[...]
    asmax = jnp.max(asb, axis=1, keepdims=True)
    lses = asmax + jnp.log(jnp.sum(jnp.exp(asb - asmax), axis=1,
                                   keepdims=True))

    ids = jax.lax.broadcasted_iota(jnp.int32, aw.shape, 1)
    tgt = jnp.min(jnp.where(aw == awmax, ids, c), axis=1, keepdims=True)

    beta = jnp.zeros_like(aw)
    for k in range(k_steps):
        nb = nb_ref[:, k, :]
        nmax = jnp.max(nb, axis=1, keepdims=True)
        en = jnp.exp(nb - nmax)
        nprob = en / jnp.sum(en, axis=1, keepdims=True)
        d = weak - nprob
        n2 = jnp.sum(d * d, axis=1, keepdims=True)
        beta = beta + jnp.exp(-n2) * nprob
    beta = beta / jnp.sum(beta, axis=1, keepdims=True)
    qd = weak - beta
    qbn = jnp.sum(qd * qd, axis=1, keepdims=True)
    lqbn = jnp.log(qbn)

    m_ref[...] = m
    lsew_ref[...] = lsew
    lses_ref[...] = lses
    lqbn_ref[...] = lqbn
    tgt_ref[...] = tgt

    @pl.when(pl.program_id(0) == 0)
    def _():
        lmax_ref[...] = jnp.full((1, 1), _NEG_INF, jnp.float32)

    bm = jnp.max(jnp.where(m > log_ct1, lqbn, _NEG_INF), axis=0,
                 keepdims=True)
    lmax_ref[...] = jnp.maximum(lmax_ref[...], bm)


def _phase_b_body(aw_ref, as_ref, m_ref, lsew_ref, lses_ref, lqbn_ref,
                  tgt_ref, lmax_ref, lt_ref, counts_ref, n_ref, s_ref,
                  *, log_ct1):
    @pl.when(pl.program_id(0) == 0)
    def _():
        counts_ref[...] = jnp.zeros_like(counts_ref)
        n_ref[...] = jnp.zeros_like(n_ref)
        s_ref[...] = jnp.zeros_like(s_ref)

    ltau = lmax_ref[...] + lt_ref[...]
    alpha = ltau - lqbn_ref[...]
    g = jnp.maximum(alpha, 1.0)
    aw = aw_ref[...]
    t = jnp.exp(g * (aw - lsew_ref[...]))
    s = jnp.sum(t, axis=1, keepdims=True)
    q = t / s
    m = m_ref[...]
    qmax = jnp.exp(g * m) / s
    mask = (m > log_ct1) & (qmax > _CT2)
    maskf = mask.astype(jnp.float32)
    lp = as_ref[...] - lses_ref[...]

    ids = jax.lax.broadcasted_iota(jnp.int32, aw.shape, 1)
    onehot = jnp.where((tgt_ref[...] == ids) & mask, 1.0, 0.0)

    counts_ref[...] += jnp.sum(onehot, axis=0, keepdims=True)
    n_ref[...] += jnp.sum(maskf, axis=0, keepdims=True)
    s_ref[...] += jnp.sum(maskf * q * lp, axis=0, keepdims=True)


def _phase_c_body(counts_ref, n_ref, s_ref, out_ref, *, c):
    counts = counts_ref[...]
    n = n_ref[...]
    freq = counts / n
    weight = jnp.where(counts > 0, 1.0 / jnp.log(_H + freq),
                       jnp.ones_like(counts))
    wsum = jnp.sum(weight, axis=1, keepdims=True)
    w_avg = weight / wsum * (wsum / c)
    row = jnp.sum(w_avg * s_ref[...], axis=1, keepdims=True)
    out_ref[...] = -row / n


def kernel(anchors_weak, anchors_strong, neighbors, eta, epoch):
    b, c = anchors_weak.shape
    k = neighbors.shape[1]
    bb_a = 256
    grid_a = b // bb_a

    f32 = jnp.float32
    row_spec = pl.BlockSpec((bb_a, 1), lambda i: (i, 0))
    mat_spec = pl.BlockSpec((bb_a, c), lambda i: (i, 0))
    scal_spec = pl.BlockSpec((1, 1), lambda i: (0, 0))
    log_ct1 = math.log(_CT1)

    m, lsew, lses, lqbn, tgt, lmax = pl.pallas_call(
        functools.partial(_phase_a_body, k_steps=k, c=c, log_ct1=log_ct1),
        grid=(grid_a,),
        in_specs=[
            mat_spec,
            mat_spec,
            pl.BlockSpec((bb_a, k, c), lambda i: (i, 0, 0)),
        ],
        out_specs=[row_spec, row_spec, row_spec, row_spec, row_spec,
                   scal_spec],
        out_shape=[
            jax.ShapeDtypeStruct((b, 1), f32),
            jax.ShapeDtypeStruct((b, 1), f32),
            jax.ShapeDtypeStruct((b, 1), f32),
            jax.ShapeDtypeStruct((b, 1), f32),
            jax.ShapeDtypeStruct((b, 1), jnp.int32),
            jax.ShapeDtypeStruct((1, 1), f32),
        ],
    )(anchors_weak, anchors_strong, neighbors)

    lt = (jnp.float32(1.0) + jnp.log(jnp.asarray(eta, f32))).reshape(1, 1)

    bb_b = 512
    grid_b = b // bb_b
    row_spec_b = pl.BlockSpec((bb_b, 1), lambda i: (i, 0))
    mat_spec_b = pl.BlockSpec((bb_b, c), lambda i: (i, 0))
    vec_spec = pl.BlockSpec((1, c), lambda i: (0, 0))
    scal_spec_b = pl.BlockSpec((1, 1), lambda i: (0, 0))

    counts, nn, svec = pl.pallas_call(
        functools.partial(_phase_b_body, log_ct1=log_ct1),
        grid=(grid_b,),
        in_specs=[mat_spec_b, mat_spec_b, row_spec_b, row_spec_b, row_spec_b,
                  row_spec_b, row_spec_b, scal_spec_b, scal_spec_b],
        out_specs=[vec_spec, scal_spec_b, vec_spec],
        out_shape=[
            jax.ShapeDtypeStruct((1, c), f32),
            jax.ShapeDtypeStruct((1, 1), f32),
            jax.ShapeDtypeStruct((1, c), f32),
        ],
    )(anchors_weak, anchors_strong, m, lsew, lses, lqbn, tgt, lmax, lt)

    loss = pl.pallas_call(
        functools.partial(_phase_c_body, c=c),
        out_shape=jax.ShapeDtypeStruct((1, 1), f32),
    )(counts, nn, svec)

    return loss[0, 0]
